# baseline (speedup 1.0000x reference)
import jax
import jax.numpy as jnp
from jax import lax
from jax.experimental import pallas as pl
from jax.experimental.pallas import tpu as pltpu
from jax.experimental.pallas import tpu_sc as plsc

GRID_SIZE = 6
NUM_ROIS = 256
NUM_VOXELS = 100000
C_IN = 64
NSAMPLE = 16
C_MID = 32
C_OUT = 32
FC = 256
N_PTS = NUM_ROIS * GRID_SIZE ** 3
N_TOT = N_PTS * NSAMPLE

SC_CORES = 2
SC_SUBCORES = 16
NW = SC_CORES * SC_SUBCORES
CHUNK = 128
N_CHUNKS = N_TOT // CHUNK
CHUNKS_PER_W = N_CHUNKS // NW

XPAD = 16


def _relu(x):
    return jnp.maximum(x, 0.0)


NRING = 4
D_ALL = C_IN // 2 + XPAD


def _sc_gather_body(idx_hbm, t_hbm, g_hbm, idx_v, bufs, sems):
    wid = lax.axis_index("s") * SC_CORES + lax.axis_index("c")
    chunk0 = wid * CHUNKS_PER_W
    pltpu.sync_copy(idx_hbm.at[pl.ds(chunk0, CHUNKS_PER_W)], idx_v)

    def start(j, r):
        pltpu.async_copy(t_hbm.at[idx_v.at[j]], bufs[r], sems[r])

    def drain_write(j, r):
        pltpu.make_async_copy(t_hbm.at[idx_v.at[j]], bufs[r], sems[r]).wait()
        pltpu.sync_copy(bufs[r], g_hbm.at[pl.ds((chunk0 + j) * CHUNK, CHUNK)])

    for r in range(NRING - 1):
        start(r, r)

    def step(jj, carry):
        j = jj * NRING
        for r in range(NRING):
            start(j + r + NRING - 1, (r + NRING - 1) % NRING)
            drain_write(j + r, r)
        return carry

    lax.fori_loop(0, CHUNKS_PER_W // NRING - 1, step, 0)
    j = CHUNKS_PER_W - NRING
    start(j + NRING - 1, NRING - 1)
    for r in range(NRING):
        drain_write(j + r, r)


def _sc_gather(idx2d, table):
    mesh = plsc.VectorSubcoreMesh(core_axis_name="c", subcore_axis_name="s")

    def body(idx_hbm, t_hbm, g_hbm, idx_v, *rest):
        _sc_gather_body(idx_hbm, t_hbm, g_hbm, idx_v,
                        rest[:NRING], rest[NRING:])

    k = pl.kernel(
        body,
        out_type=jax.ShapeDtypeStruct((N_TOT, D_ALL), jnp.int32),
        mesh=mesh,
        scratch_types=[pltpu.VMEM((CHUNKS_PER_W, CHUNK), jnp.int32)]
        + [pltpu.VMEM((CHUNK, D_ALL), jnp.int32)] * NRING
        + [pltpu.SemaphoreType.DMA] * NRING,
        compiler_params=pltpu.CompilerParams(use_tc_tiling_on_sc=False),
    )
    return k(idx2d, table)


def _pool_body(g_ref, gz_ref, wfe_ref, wfo_ref, wp_ref, w2_ref, out_ref):
    wfe = wfe_ref[...]
    wfo = wfo_ref[...]
    wp = wp_ref[...]
    w2 = w2_ref[...]
    gz = gz_ref[...]
    acc = None
    for s in range(NSAMPLE):
        gs = g_ref[s]
        fw = gs[:, :C_IN // 2]
        f_even = jax.lax.bitcast_convert_type(
            jax.lax.shift_left(fw, 16), jnp.float32)
        f_odd = jax.lax.bitcast_convert_type(
            jax.lax.bitwise_and(fw, jnp.int32(-65536)), jnp.float32)
        gx = jax.lax.bitcast_convert_type(gs[:, C_IN // 2:], jnp.float32)
        hf = (jnp.dot(f_even, wfe, preferred_element_type=jnp.float32)
              + jnp.dot(f_odd, wfo, preferred_element_type=jnp.float32))
        hp = jnp.dot(gx - gz, wp, preferred_element_type=jnp.float32)
        h = _relu(hf + hp)
        h2 = _relu(jnp.dot(h, w2, preferred_element_type=jnp.float32))
        acc = h2 if acc is None else jnp.maximum(acc, h2)
    out_ref[...] = acc


def _pool(g3, gxyzp, wfe, wfo, wpp, w2):
    nblk = 32
    rows = N_PTS // nblk
    return pl.pallas_call(
        _pool_body,
        grid=(nblk,),
        in_specs=[
            pl.BlockSpec((NSAMPLE, rows, D_ALL), lambda i: (0, i, 0)),
            pl.BlockSpec((rows, XPAD), lambda i: (i, 0)),
            pl.BlockSpec((C_IN // 2, C_MID), lambda i: (0, 0)),
            pl.BlockSpec((C_IN // 2, C_MID), lambda i: (0, 0)),
            pl.BlockSpec((XPAD, C_MID), lambda i: (0, 0)),
            pl.BlockSpec((C_MID, C_OUT), lambda i: (0, 0)),
        ],
        out_specs=pl.BlockSpec((rows, C_OUT), lambda i: (i, 0)),
        out_shape=jax.ShapeDtypeStruct((N_PTS, C_OUT), jnp.float32),
    )(g3, gxyzp, wfe, wfo, wpp, w2)


def _head_body(x_ref, s1_ref, s2_ref, c1_ref, c2_ref, co_ref, bc_ref,
               i1_ref, i2_ref, io_ref, bi_ref, r1_ref, r2_ref, ro_ref, br_ref,
               cls_ref, iou_ref, reg_ref):
    x = _relu(jnp.dot(x_ref[...], s1_ref[...], preferred_element_type=jnp.float32))
    x = _relu(jnp.dot(x, s2_ref[...], preferred_element_type=jnp.float32))

    def branch(w1, w2, wo):
        h = _relu(jnp.dot(x, w1[...], preferred_element_type=jnp.float32))
        h = _relu(jnp.dot(h, w2[...], preferred_element_type=jnp.float32))
        return jnp.dot(h, wo[...], preferred_element_type=jnp.float32)

    cls_ref[...] = branch(c1_ref, c2_ref, co_ref) + bc_ref[...]
    iou_ref[...] = branch(i1_ref, i2_ref, io_ref) + bi_ref[...]
    reg_ref[...] = branch(r1_ref, r2_ref, ro_ref) + br_ref[...]


def _head(x, s1, s2, c1, c2, co, bc, i1, i2, io, bi, r1, r2, ro, br):
    pre = GRID_SIZE ** 3 * C_OUT
    full = lambda shape: pl.BlockSpec(shape, lambda: tuple(0 for _ in shape))
    return pl.pallas_call(
        _head_body,
        in_specs=[
            full((NUM_ROIS, pre)),
            full((pre, FC)), full((FC, FC)),
            full((FC, FC)), full((FC, FC)), full((FC, 1)), full((1, 1)),
            full((FC, FC)), full((FC, FC)), full((FC, 1)), full((1, 1)),
            full((FC, FC)), full((FC, FC)), full((FC, 7)), full((1, 7)),
        ],
        out_specs=[full((NUM_ROIS, 1)), full((NUM_ROIS, 1)), full((NUM_ROIS, 7))],
        out_shape=[
            jax.ShapeDtypeStruct((NUM_ROIS, 1), jnp.float32),
            jax.ShapeDtypeStruct((NUM_ROIS, 1), jnp.float32),
            jax.ShapeDtypeStruct((NUM_ROIS, 7), jnp.float32),
        ],
    )(x, s1, s2, c1, c2, co, bc.reshape(1, 1), i1, i2, io, bi.reshape(1, 1),
      r1, r2, ro, br.reshape(1, 7))


def _grid_points(rois):
    gi = jnp.arange(GRID_SIZE, dtype=jnp.float32)
    dense_idx = jnp.stack(
        jnp.meshgrid(gi, gi, gi, indexing='ij'), axis=-1).reshape(-1, 3)
    lwh = rois[:, 3:6]
    local = (dense_idx[None, :, :] + 0.5) / GRID_SIZE * lwh[:, None, :] \
        - lwh[:, None, :] / 2.0
    angle = rois[:, 6]
    cosa = jnp.cos(angle)
    sina = jnp.sin(angle)
    zeros = jnp.zeros_like(angle)
    ones = jnp.ones_like(angle)
    rot = jnp.stack([cosa, sina, zeros, -sina, cosa, zeros, zeros, zeros,
                     ones], axis=1).reshape(-1, 3, 3)
    pts = jnp.einsum('npc,ncd->npd', local, rot)
    return (pts + rois[:, None, 0:3]).reshape(-1, 3)


def kernel(rois, voxel_xyz, voxel_features, neighbor_idx, W_feat, W_pos,
           W_mlp2, W_s1, W_s2, W_c1, W_c2, W_c_out, b_c_out, W_i1, W_i2,
           W_i_out, b_i_out, W_r1, W_r2, W_r_out, b_r_out):
    xyzp = jnp.pad(voxel_xyz, ((0, 0), (0, XPAD - 3)))
    wpp = jnp.pad(W_pos, ((0, XPAD - 3), (0, 0)))
    idx2d = neighbor_idx.T.reshape(N_CHUNKS, CHUNK)
    fb = voxel_features.astype(jnp.bfloat16).reshape(NUM_VOXELS, C_IN // 2, 2)
    fi = jax.lax.bitcast_convert_type(fb, jnp.int32)
    xi = jax.lax.bitcast_convert_type(xyzp, jnp.int32)
    table = jnp.concatenate([fi, xi], axis=1)

    g = _sc_gather(idx2d, table)
    g3 = g.reshape(NSAMPLE, N_PTS, D_ALL)

    grid_xyz = _grid_points(rois)
    gxyzp = jnp.pad(grid_xyz, ((0, 0), (0, XPAD - 3)))

    pooled = _pool(g3, gxyzp, W_feat[0::2], W_feat[1::2], wpp, W_mlp2)

    x = pooled.reshape(NUM_ROIS, GRID_SIZE ** 3 * C_OUT)
    cls, iou, reg = _head(x, W_s1, W_s2, W_c1, W_c2, W_c_out, b_c_out,
                          W_i1, W_i2, W_i_out, b_i_out,
                          W_r1, W_r2, W_r_out, b_r_out)
    return (cls, iou, reg)


# speedup vs baseline: 1.1100x; 1.1100x over previous
import jax
import jax.numpy as jnp
from jax import lax
from jax.experimental import pallas as pl
from jax.experimental.pallas import tpu as pltpu
from jax.experimental.pallas import tpu_sc as plsc

GRID_SIZE = 6
NUM_ROIS = 256
NUM_VOXELS = 100000
C_IN = 64
NSAMPLE = 16
C_MID = 32
C_OUT = 32
FC = 256
N_PTS = NUM_ROIS * GRID_SIZE ** 3
N_TOT = N_PTS * NSAMPLE

SC_CORES = 2
SC_SUBCORES = 16
NW = SC_CORES * SC_SUBCORES
CHUNK = 128
N_CHUNKS = N_TOT // CHUNK
CHUNKS_PER_W = N_CHUNKS // NW

XPAD = 16


def _relu(x):
    return jnp.maximum(x, 0.0)


NRING = 4
D_ALL = C_IN + XPAD


def _sc_gather_body(n_chunks_w, idx_hbm, t_hbm, g_hbm, idx_v, bufs, sems):
    wid = lax.axis_index("s") * SC_CORES + lax.axis_index("c")
    chunk0 = wid * n_chunks_w
    pltpu.sync_copy(idx_hbm.at[pl.ds(chunk0, n_chunks_w)], idx_v)

    def start(j, r):
        pltpu.async_copy(t_hbm.at[idx_v.at[j]], bufs[r], sems[r])

    def drain_write(j, r):
        pltpu.make_async_copy(t_hbm.at[idx_v.at[j]], bufs[r], sems[r]).wait()
        pltpu.sync_copy(bufs[r], g_hbm.at[pl.ds((chunk0 + j) * CHUNK, CHUNK)])

    for r in range(NRING - 1):
        start(r, r)

    def step(jj, carry):
        j = jj * NRING
        for r in range(NRING):
            start(j + r + NRING - 1, (r + NRING - 1) % NRING)
            drain_write(j + r, r)
        return carry

    lax.fori_loop(0, n_chunks_w // NRING - 1, step, 0)
    j = n_chunks_w - NRING
    start(j + NRING - 1, NRING - 1)
    for r in range(NRING):
        drain_write(j + r, r)


def _sc_gather(idx2d, table):
    n_chunks = idx2d.shape[0]
    n_chunks_w = n_chunks // NW
    mesh = plsc.VectorSubcoreMesh(core_axis_name="c", subcore_axis_name="s")

    def body(idx_hbm, t_hbm, g_hbm, idx_v, *rest):
        _sc_gather_body(n_chunks_w, idx_hbm, t_hbm, g_hbm, idx_v,
                        rest[:NRING], rest[NRING:])

    k = pl.kernel(
        body,
        out_type=jax.ShapeDtypeStruct((n_chunks * CHUNK, D_ALL), jnp.float32),
        mesh=mesh,
        scratch_types=[pltpu.VMEM((n_chunks_w, CHUNK), jnp.int32)]
        + [pltpu.VMEM((CHUNK, D_ALL), jnp.float32)] * NRING
        + [pltpu.SemaphoreType.DMA] * NRING,
        compiler_params=pltpu.CompilerParams(use_tc_tiling_on_sc=False),
    )
    return k(idx2d, table)


def _pool_body(g_ref, gz_ref, wf_ref, wp_ref, w2_ref, out_ref):
    wf = wf_ref[...]
    wp = wp_ref[...]
    w2 = w2_ref[...]
    gz = gz_ref[...]
    acc = None
    for s in range(NSAMPLE):
        gs = g_ref[s]
        hf = jnp.dot(gs[:, :C_IN], wf, preferred_element_type=jnp.float32)
        hp = jnp.dot(gs[:, C_IN:] - gz, wp, preferred_element_type=jnp.float32)
        h = _relu(hf + hp)
        h2 = _relu(jnp.dot(h, w2, preferred_element_type=jnp.float32))
        acc = h2 if acc is None else jnp.maximum(acc, h2)
    out_ref[...] = acc


def _pool(g3, gxyzp, wf, wpp, w2):
    rows = 1728
    nblk = g3.shape[1] // rows
    return pl.pallas_call(
        _pool_body,
        grid=(nblk,),
        in_specs=[
            pl.BlockSpec((NSAMPLE, rows, D_ALL), lambda i: (0, i, 0)),
            pl.BlockSpec((rows, XPAD), lambda i: (i, 0)),
            pl.BlockSpec((C_IN, C_MID), lambda i: (0, 0)),
            pl.BlockSpec((XPAD, C_MID), lambda i: (0, 0)),
            pl.BlockSpec((C_MID, C_OUT), lambda i: (0, 0)),
        ],
        out_specs=pl.BlockSpec((rows, C_OUT), lambda i: (i, 0)),
        out_shape=jax.ShapeDtypeStruct((g3.shape[1], C_OUT), jnp.float32),
    )(g3, gxyzp, wf, wpp, w2)


def _head_body(x_ref, s1_ref, s2_ref, c1_ref, c2_ref, co_ref, bc_ref,
               i1_ref, i2_ref, io_ref, bi_ref, r1_ref, r2_ref, ro_ref, br_ref,
               cls_ref, iou_ref, reg_ref):
    x = _relu(jnp.dot(x_ref[...], s1_ref[...], preferred_element_type=jnp.float32))
    x = _relu(jnp.dot(x, s2_ref[...], preferred_element_type=jnp.float32))

    def branch(w1, w2, wo):
        h = _relu(jnp.dot(x, w1[...], preferred_element_type=jnp.float32))
        h = _relu(jnp.dot(h, w2[...], preferred_element_type=jnp.float32))
        return jnp.dot(h, wo[...], preferred_element_type=jnp.float32)

    cls_ref[...] = branch(c1_ref, c2_ref, co_ref) + bc_ref[...]
    iou_ref[...] = branch(i1_ref, i2_ref, io_ref) + bi_ref[...]
    reg_ref[...] = branch(r1_ref, r2_ref, ro_ref) + br_ref[...]


def _head(x, s1, s2, c1, c2, co, bc, i1, i2, io, bi, r1, r2, ro, br):
    pre = GRID_SIZE ** 3 * C_OUT
    full = lambda shape: pl.BlockSpec(shape, lambda: tuple(0 for _ in shape))
    return pl.pallas_call(
        _head_body,
        in_specs=[
            full((NUM_ROIS, pre)),
            full((pre, FC)), full((FC, FC)),
            full((FC, FC)), full((FC, FC)), full((FC, 1)), full((1, 1)),
            full((FC, FC)), full((FC, FC)), full((FC, 1)), full((1, 1)),
            full((FC, FC)), full((FC, FC)), full((FC, 7)), full((1, 7)),
        ],
        out_specs=[full((NUM_ROIS, 1)), full((NUM_ROIS, 1)), full((NUM_ROIS, 7))],
        out_shape=[
            jax.ShapeDtypeStruct((NUM_ROIS, 1), jnp.float32),
            jax.ShapeDtypeStruct((NUM_ROIS, 1), jnp.float32),
            jax.ShapeDtypeStruct((NUM_ROIS, 7), jnp.float32),
        ],
    )(x, s1, s2, c1, c2, co, bc.reshape(1, 1), i1, i2, io, bi.reshape(1, 1),
      r1, r2, ro, br.reshape(1, 7))


def _grid_points(rois):
    gi = jnp.arange(GRID_SIZE, dtype=jnp.float32)
    dense_idx = jnp.stack(
        jnp.meshgrid(gi, gi, gi, indexing='ij'), axis=-1).reshape(-1, 3)
    lwh = rois[:, 3:6]
    local = (dense_idx[None, :, :] + 0.5) / GRID_SIZE * lwh[:, None, :] \
        - lwh[:, None, :] / 2.0
    angle = rois[:, 6]
    cosa = jnp.cos(angle)
    sina = jnp.sin(angle)
    zeros = jnp.zeros_like(angle)
    ones = jnp.ones_like(angle)
    rot = jnp.stack([cosa, sina, zeros, -sina, cosa, zeros, zeros, zeros,
                     ones], axis=1).reshape(-1, 3, 3)
    pts = jnp.einsum('npc,ncd->npd', local, rot)
    return (pts + rois[:, None, 0:3]).reshape(-1, 3)


def kernel(rois, voxel_xyz, voxel_features, neighbor_idx, W_feat, W_pos,
           W_mlp2, W_s1, W_s2, W_c1, W_c2, W_c_out, b_c_out, W_i1, W_i2,
           W_i_out, b_i_out, W_r1, W_r2, W_r_out, b_r_out):
    xyzp = jnp.pad(voxel_xyz, ((0, 0), (0, XPAD - 3)))
    wpp = jnp.pad(W_pos, ((0, XPAD - 3), (0, 0)))
    idx2d = neighbor_idx.T.reshape(N_CHUNKS, CHUNK)
    table = jnp.concatenate([voxel_features, xyzp], axis=1)

    grid_xyz = _grid_points(rois)
    gxyzp = jnp.pad(grid_xyz, ((0, 0), (0, XPAD - 3)))

    half = N_PTS // 2
    idx4 = idx2d.reshape(NSAMPLE, N_CHUNKS // NSAMPLE, CHUNK)
    hc = idx4.shape[1] // 2
    pooled_halves = []
    gs = [
        _sc_gather(idx4[:, :hc].reshape(-1, CHUNK), table),
        _sc_gather(idx4[:, hc:].reshape(-1, CHUNK), table),
    ]
    for i, g in enumerate(gs):
        g3 = g.reshape(NSAMPLE, half, D_ALL)
        pooled_halves.append(
            _pool(g3, gxyzp[i * half:(i + 1) * half], W_feat, wpp, W_mlp2))
    x = jnp.concatenate(pooled_halves, axis=0)
    x = x.reshape(NUM_ROIS, GRID_SIZE ** 3 * C_OUT)
    cls, iou, reg = _head(x, W_s1, W_s2, W_c1, W_c2, W_c_out, b_c_out,
                          W_i1, W_i2, W_i_out, b_i_out,
                          W_r1, W_r2, W_r_out, b_r_out)
    return (cls, iou, reg)
